# baseline (device time: 41334 ns/iter reference)
import jax
import jax.numpy as jnp
from jax import lax
from jax.experimental import pallas as pl
from jax.experimental.pallas import tpu as pltpu

N_Z = 4


def kernel(Q, K, V):
    b, sq, h, d = Q.shape
    scale = d ** -0.5
    lanes = 128

    def body(q_ref, k_ref, v_ref, out_ref, comm_ref, send_sems, recv_sems):
        my_x = lax.axis_index("x")
        my_y = lax.axis_index("y")
        my_z = lax.axis_index("z")

        barrier = pltpu.get_barrier_semaphore()
        for dz in (1, 2, 3):
            pl.semaphore_signal(
                barrier,
                inc=1,
                device_id=(my_x, my_y, (my_z + dz) % N_Z),
                device_id_type=pl.DeviceIdType.MESH,
            )

        q = q_ref[...]
        k = k_ref[...]
        v = v_ref[...]
        s = jnp.sum(q * k, axis=-1) * scale
        m = jnp.max(s, axis=1, keepdims=True)
        p = jnp.exp(s - m)
        l = jnp.sum(p, axis=1, keepdims=True)
        o = jnp.sum(p[..., None] * v, axis=1)
        packed = jnp.concatenate(
            [
                o,
                m[:, 0, :, None],
                l[:, 0, :, None],
                jnp.zeros((b, h, lanes - d - 2), jnp.float32),
            ],
            axis=-1,
        )
        comm_ref[0] = packed

        pl.semaphore_wait(barrier, 3)

        rdmas = []
        for dz in (1, 2, 3):
            rdma = pltpu.make_async_remote_copy(
                src_ref=comm_ref.at[0],
                dst_ref=comm_ref.at[dz],
                send_sem=send_sems.at[dz],
                recv_sem=recv_sems.at[dz],
                device_id=(my_x, my_y, (my_z + dz) % N_Z),
                device_id_type=pl.DeviceIdType.MESH,
            )
            rdma.start()
            rdmas.append(rdma)
        for rdma in rdmas:
            rdma.wait_recv()

        allp = comm_ref[...]
        o_all = allp[..., :d]
        m_all = allp[..., d]
        l_all = allp[..., d + 1]
        mg = jnp.max(m_all, axis=0)
        w = jnp.exp(m_all - mg[None])
        lg = jnp.sum(l_all * w, axis=0)
        og = jnp.sum(o_all * w[..., None], axis=0)
        res = og / lg[..., None]
        out_ref[...] = res[:, None]

        for rdma in rdmas:
            rdma.wait_send()

    return pl.pallas_call(
        body,
        out_shape=jax.ShapeDtypeStruct((b, sq, h, d), jnp.float32),
        in_specs=[pl.BlockSpec(memory_space=pltpu.VMEM)] * 3,
        out_specs=pl.BlockSpec(memory_space=pltpu.VMEM),
        scratch_shapes=[
            pltpu.VMEM((N_Z, b, h, lanes), jnp.float32),
            pltpu.SemaphoreType.DMA((N_Z,)),
            pltpu.SemaphoreType.DMA((N_Z,)),
        ],
        compiler_params=pltpu.CompilerParams(collective_id=0),
    )(Q, K, V)


# device time: 21402 ns/iter; 1.9313x vs baseline; 1.9313x over previous
import jax
import jax.numpy as jnp
from jax import lax
from jax.experimental import pallas as pl
from jax.experimental.pallas import tpu as pltpu

N_Z = 4
LANES = 640


def kernel(Q, K, V):
    b, sq, h, d = Q.shape
    skv = K.shape[1]
    hd = h * d
    scale = d ** -0.5

    Qf = Q.reshape(b, hd)
    Kf = K.reshape(b, skv, hd)
    Vf = V.reshape(b, skv, hd)

    def body(q_ref, k_ref, v_ref, out_ref, comm_ref, send_sems, recv_sems):
        my_x = lax.axis_index("x")
        my_y = lax.axis_index("y")
        my_z = lax.axis_index("z")

        barrier = pltpu.get_barrier_semaphore()
        for dz in (1, 2, 3):
            pl.semaphore_signal(
                barrier,
                inc=1,
                device_id=(my_x, my_y, (my_z + dz) % N_Z),
                device_id_type=pl.DeviceIdType.MESH,
            )

        q = q_ref[...]
        k = k_ref[...].astype(jnp.bfloat16)
        v = v_ref[...].astype(jnp.bfloat16)

        row = lax.broadcasted_iota(jnp.int32, (hd, h), 0) // d
        col = lax.broadcasted_iota(jnp.int32, (hd, h), 1)
        mask1 = (row == col).astype(jnp.float32)
        qbd = (q[:, :, None] * mask1[None]).astype(jnp.bfloat16)

        s = lax.dot_general(
            k, qbd, (((2,), (1,)), ((0,), (0,))),
            preferred_element_type=jnp.float32,
        ) * scale
        m = jnp.max(s, axis=1)
        p = jnp.exp(s - m[:, None, :])
        l = jnp.sum(p, axis=1)

        r = lax.dot_general(
            p.astype(jnp.bfloat16), v, (((1,), (1,)), ((0,), (0,))),
            preferred_element_type=jnp.float32,
        )
        hrow = lax.broadcasted_iota(jnp.int32, (h, hd), 0)
        hcol = lax.broadcasted_iota(jnp.int32, (h, hd), 1) // d
        mask2 = (hrow == hcol).astype(jnp.float32)
        o_flat = jnp.sum(r * mask2[None], axis=1)

        packed = jnp.concatenate(
            [o_flat, m, l, jnp.zeros((b, LANES - hd - 2 * h), jnp.float32)],
            axis=-1,
        )
        comm_ref[0] = packed

        pl.semaphore_wait(barrier, 3)

        rdmas = []
        for dz in (1, 2, 3):
            rdma = pltpu.make_async_remote_copy(
                src_ref=comm_ref.at[0],
                dst_ref=comm_ref.at[dz],
                send_sem=send_sems.at[dz],
                recv_sem=recv_sems.at[dz],
                device_id=(my_x, my_y, (my_z + dz) % N_Z),
                device_id_type=pl.DeviceIdType.MESH,
            )
            rdma.start()
            rdmas.append(rdma)
        for rdma in rdmas:
            rdma.wait_recv()

        allp = comm_ref[...]
        o_all = allp[..., :hd]
        m_all = allp[..., hd:hd + h]
        l_all = allp[..., hd + h:hd + 2 * h]
        mg = jnp.max(m_all, axis=0)
        w = jnp.exp(m_all - mg[None])
        lg = jnp.sum(l_all * w, axis=0)
        w_exp = jnp.dot(
            w.reshape(N_Z * b, h), mask2, preferred_element_type=jnp.float32
        ).reshape(N_Z, b, hd)
        lg_exp = jnp.dot(lg, mask2, preferred_element_type=jnp.float32)
        og = jnp.sum(o_all * w_exp, axis=0)
        out_ref[...] = og / lg_exp

        for rdma in rdmas:
            rdma.wait_send()

    out = pl.pallas_call(
        body,
        out_shape=jax.ShapeDtypeStruct((b, hd), jnp.float32),
        in_specs=[pl.BlockSpec(memory_space=pltpu.VMEM)] * 3,
        out_specs=pl.BlockSpec(memory_space=pltpu.VMEM),
        scratch_shapes=[
            pltpu.VMEM((N_Z, b, LANES), jnp.float32),
            pltpu.SemaphoreType.DMA((N_Z,)),
            pltpu.SemaphoreType.DMA((N_Z,)),
        ],
        compiler_params=pltpu.CompilerParams(collective_id=0),
    )(Qf, Kf, Vf)
    return out.reshape(b, sq, h, d)


# device time: 20998 ns/iter; 1.9685x vs baseline; 1.0192x over previous
import jax
import jax.numpy as jnp
from jax import lax
from jax.experimental import pallas as pl
from jax.experimental.pallas import tpu as pltpu

N_Z = 4
LANES = 640


def kernel(Q, K, V):
    b, sq, h, d = Q.shape
    skv = K.shape[1]
    hd = h * d
    scale = d ** -0.5

    Qf = Q.reshape(b, hd)
    Kf = K.reshape(b, skv, hd)
    Vf = V.reshape(b, skv, hd)

    def body(q_ref, k_ref, v_ref, out_ref, comm_ref, send_sems, recv_sems):
        my_x = lax.axis_index("x")
        my_y = lax.axis_index("y")
        my_z = lax.axis_index("z")

        barrier = pltpu.get_barrier_semaphore()
        for dz in (1, 2, 3):
            pl.semaphore_signal(
                barrier,
                inc=1,
                device_id=(my_x, my_y, (my_z + dz) % N_Z),
                device_id_type=pl.DeviceIdType.MESH,
            )

        q = q_ref[...]
        k = k_ref[...].astype(jnp.bfloat16)
        v = v_ref[...].astype(jnp.bfloat16)

        row = lax.broadcasted_iota(jnp.int32, (hd, h), 0) // d
        col = lax.broadcasted_iota(jnp.int32, (hd, h), 1)
        mask1 = (row == col).astype(jnp.float32)
        qbd = (q[:, :, None] * mask1[None]).astype(jnp.bfloat16)

        s = lax.dot_general(
            k, qbd, (((2,), (1,)), ((0,), (0,))),
            preferred_element_type=jnp.float32,
        ) * scale
        st = jnp.swapaxes(s, 1, 2)
        m = jnp.max(st, axis=2)
        p = jnp.exp(st - m[:, :, None])
        l = jnp.sum(p, axis=2)

        r = lax.dot_general(
            p.astype(jnp.bfloat16), v, (((2,), (1,)), ((0,), (0,))),
            preferred_element_type=jnp.float32,
        )
        hrow = lax.broadcasted_iota(jnp.int32, (h, hd), 0)
        hcol = lax.broadcasted_iota(jnp.int32, (h, hd), 1) // d
        mask2 = (hrow == hcol).astype(jnp.float32)
        o_flat = jnp.sum(r * mask2[None], axis=1)

        packed = jnp.concatenate(
            [o_flat, m, l, jnp.zeros((b, LANES - hd - 2 * h), jnp.float32)],
            axis=-1,
        )
        comm_ref[0] = packed

        pl.semaphore_wait(barrier, 3)

        rdmas = []
        for dz in (1, 2, 3):
            rdma = pltpu.make_async_remote_copy(
                src_ref=comm_ref.at[0],
                dst_ref=comm_ref.at[dz],
                send_sem=send_sems.at[dz],
                recv_sem=recv_sems.at[dz],
                device_id=(my_x, my_y, (my_z + dz) % N_Z),
                device_id_type=pl.DeviceIdType.MESH,
            )
            rdma.start()
            rdmas.append(rdma)
        for rdma in rdmas:
            rdma.wait_recv()

        allp = comm_ref[...]
        o_all = allp[..., :hd]
        m_all = allp[..., hd:hd + h]
        l_all = allp[..., hd + h:hd + 2 * h]
        mg = jnp.max(m_all, axis=0)
        w = jnp.exp(m_all - mg[None])
        lg = jnp.sum(l_all * w, axis=0)
        w_exp = jnp.dot(
            w.reshape(N_Z * b, h), mask2, preferred_element_type=jnp.float32
        ).reshape(N_Z, b, hd)
        lg_exp = jnp.dot(lg, mask2, preferred_element_type=jnp.float32)
        og = jnp.sum(o_all * w_exp, axis=0)
        out_ref[...] = og / lg_exp

        for rdma in rdmas:
            rdma.wait_send()

    out = pl.pallas_call(
        body,
        out_shape=jax.ShapeDtypeStruct((b, hd), jnp.float32),
        in_specs=[pl.BlockSpec(memory_space=pltpu.VMEM)] * 3,
        out_specs=pl.BlockSpec(memory_space=pltpu.VMEM),
        scratch_shapes=[
            pltpu.VMEM((N_Z, b, LANES), jnp.float32),
            pltpu.SemaphoreType.DMA((N_Z,)),
            pltpu.SemaphoreType.DMA((N_Z,)),
        ],
        compiler_params=pltpu.CompilerParams(collective_id=0),
    )(Qf, Kf, Vf)
    return out.reshape(b, sq, h, d)
